# Initial kernel scaffold; baseline (speedup 1.0000x reference)
import jax
import jax.numpy as jnp
from jax.experimental import pallas as pl


def kernel(x, upsamples):
    raise NotImplementedError("write your pallas kernel here")


# speedup vs baseline: 1.3034x; 1.3034x over previous
import functools

import jax
import jax.numpy as jnp
from jax import lax
from jax.experimental import pallas as pl
from jax.experimental.pallas import tpu as pltpu
from jax.experimental.pallas import tpu_sc as plsc

_N_COARSE = 50000
_N_FINE = 100000
_D = 128
_CH = 128
_NW = 32
_NFULL = _N_FINE // _CH
_TAIL = _N_FINE - _NFULL * _CH
_NCHUNKS = _NFULL + 1
_MAXC = -(-_NCHUNKS // _NW)

_mesh = plsc.VectorSubcoreMesh(core_axis_name="c", subcore_axis_name="s")


@functools.partial(
    pl.kernel,
    out_type=jax.ShapeDtypeStruct((_N_FINE, _D), jnp.float32),
    mesh=_mesh,
    scratch_types=[
        pltpu.VMEM((_CH,), jnp.int32),
        pltpu.VMEM((_CH, _D), jnp.float32),
        pltpu.VMEM((_TAIL,), jnp.int32),
        pltpu.VMEM((_TAIL, _D), jnp.float32),
        pltpu.SemaphoreType.DMA,
    ],
)
def _sc_gather(xp_hbm, idx_hbm, out_hbm, idx_v, rows_v, idx_t, rows_t, sem):
    wid = lax.axis_index("s") * 2 + lax.axis_index("c")

    def body(c, carry):
        chunk = wid + c * _NW
        base = chunk * _CH

        @pl.when(chunk < _NFULL)
        def _():
            pltpu.sync_copy(idx_hbm.at[pl.ds(base, _CH)], idx_v)
            pltpu.async_copy(xp_hbm.at[idx_v], rows_v, sem).wait()
            pltpu.sync_copy(rows_v, out_hbm.at[pl.ds(base, _CH)])

        @pl.when(chunk == _NFULL)
        def _():
            pltpu.sync_copy(idx_hbm.at[pl.ds(base, _TAIL)], idx_t)
            pltpu.async_copy(xp_hbm.at[idx_t], rows_t, sem).wait()
            pltpu.sync_copy(rows_t, out_hbm.at[pl.ds(base, _TAIL)])

        return carry

    lax.fori_loop(0, _MAXC, body, 0)


def kernel(x, upsamples):
    xp = jnp.concatenate([x, jnp.zeros_like(x[:1, :])], axis=0)
    idx = upsamples[:, 0]
    return _sc_gather(xp, idx)


# speedup vs baseline: 1.5868x; 1.2175x over previous
import functools

import jax
import jax.numpy as jnp
from jax import lax
from jax.experimental import pallas as pl
from jax.experimental.pallas import tpu as pltpu
from jax.experimental.pallas import tpu_sc as plsc

_N_COARSE = 50000
_N_FINE = 100000
_D = 128
_CH = 128
_NW = 32
_NFULL = _N_FINE // _CH
_TAIL = _N_FINE - _NFULL * _CH
_NPAIRS = 12
_LASTFULL = _NFULL - 1

_mesh = plsc.VectorSubcoreMesh(core_axis_name="c", subcore_axis_name="s")


@functools.partial(
    pl.kernel,
    out_type=jax.ShapeDtypeStruct((_N_FINE, _D), jnp.float32),
    mesh=_mesh,
    scratch_types=[
        pltpu.VMEM((2, _CH), jnp.int32),
        pltpu.VMEM((2, _CH, _D), jnp.float32),
        pltpu.VMEM((_TAIL,), jnp.int32),
        pltpu.VMEM((_TAIL, _D), jnp.float32),
        pltpu.SemaphoreType.DMA,
        pltpu.SemaphoreType.DMA,
        pltpu.SemaphoreType.DMA,
        pltpu.SemaphoreType.DMA,
        pltpu.SemaphoreType.DMA,
        pltpu.SemaphoreType.DMA,
        pltpu.SemaphoreType.DMA,
    ],
)
def _sc_gather(xp_hbm, idx_hbm, out_hbm, idx_v, rows_v, idx_t, rows_t,
               si0, si1, sg0, sg1, sw0, sw1, st):
    wid = lax.axis_index("s") * 2 + lax.axis_index("c")
    nc = jnp.where(wid <= 12, 25, 24)

    sem_i = (si0, si1)
    sem_g = (sg0, sg1)
    sem_w = (sw0, sw1)

    def chunk_step(c, b):
        chunk = wid + c * _NW
        base = chunk * _CH
        my_idx = idx_v.at[b]
        my_rows = rows_v.at[b]

        @pl.when(c == 0)
        def _():
            pltpu.async_copy(idx_hbm.at[pl.ds(base, _CH)], my_idx, sem_i[b])
            pltpu.async_copy(
                idx_hbm.at[pl.ds(base + _NW * _CH, _CH)], idx_v.at[1 - b],
                sem_i[1 - b])

        pltpu.make_async_copy(idx_hbm.at[pl.ds(0, _CH)], my_idx,
                              sem_i[b]).wait()

        @pl.when(c >= 2)
        def _():
            pltpu.make_async_copy(my_rows, out_hbm.at[pl.ds(0, _CH)],
                                  sem_w[b]).wait()

        gather = pltpu.async_copy(xp_hbm.at[my_idx], my_rows, sem_g[b])
        gather.wait()

        @pl.when(c + 2 < nc)
        def _():
            pltpu.async_copy(
                idx_hbm.at[pl.ds(base + 2 * _NW * _CH, _CH)], my_idx,
                sem_i[b])

        pltpu.async_copy(my_rows, out_hbm.at[pl.ds(base, _CH)], sem_w[b])

    def pair_body(p, carry):
        chunk_step(2 * p, 0)
        chunk_step(2 * p + 1, 1)
        return carry

    lax.fori_loop(0, _NPAIRS, pair_body, 0)

    @pl.when(nc == 25)
    def _():
        chunk_step(jnp.int32(24), 0)

    pltpu.make_async_copy(rows_v.at[0], out_hbm.at[pl.ds(0, _CH)], sw0).wait()
    pltpu.make_async_copy(rows_v.at[1], out_hbm.at[pl.ds(0, _CH)], sw1).wait()

    @pl.when(wid == _NW - 1)
    def _():
        tbase = _NFULL * _CH
        pltpu.sync_copy(idx_hbm.at[pl.ds(tbase, _TAIL)], idx_t)
        pltpu.async_copy(xp_hbm.at[idx_t], rows_t, st).wait()
        pltpu.sync_copy(rows_t, out_hbm.at[pl.ds(tbase, _TAIL)])


def kernel(x, upsamples):
    xp = jnp.concatenate([x, jnp.zeros_like(x[:1, :])], axis=0)
    idx = upsamples[:, 0]
    return _sc_gather(xp, idx)


# speedup vs baseline: 1.9333x; 1.2183x over previous
import functools

import jax
import jax.numpy as jnp
from jax import lax
from jax.experimental import pallas as pl
from jax.experimental.pallas import tpu as pltpu
from jax.experimental.pallas import tpu_sc as plsc

_N_COARSE = 50000
_N_FINE = 100000
_D = 128
_CH = 128
_NW = 32
_NFULL = _N_FINE // _CH
_TAIL = _N_FINE - _NFULL * _CH
_NPAIRS = 12
_L = 16

_mesh = plsc.VectorSubcoreMesh(core_axis_name="c", subcore_axis_name="s")


def _clamp_detect(idx_ref, save_ref, n):
    clamp = jnp.full((_L,), _N_COARSE - 1, jnp.int32)
    one = jnp.ones((_L,), jnp.int32)
    zero = jnp.zeros((_L,), jnp.int32)
    shadow = zero
    for j in range(n // _L):
        v = idx_ref[pl.ds(j * _L, _L)]
        save_ref[pl.ds(j * _L, _L)] = v
        shadow = shadow + jnp.where(v >= _N_COARSE, one, zero)
        idx_ref[pl.ds(j * _L, _L)] = jnp.minimum(v, clamp)
    total = shadow[0]
    for l in range(1, _L):
        total = total + shadow[l]
    return total > 0


def _zero_shadow_rows(save_ref, rows_ref, n):
    zero = jnp.zeros((_L,), jnp.float32)

    def body(i, carry):
        orig = save_ref[pl.ds(i, _L)][0]

        @pl.when(orig == _N_COARSE)
        def _():
            for k in range(_D // _L):
                rows_ref[i, pl.ds(k * _L, _L)] = zero
        return carry

    lax.fori_loop(0, n, body, 0)


@functools.partial(
    pl.kernel,
    out_type=jax.ShapeDtypeStruct((_N_FINE, _D), jnp.float32),
    mesh=_mesh,
    scratch_types=[
        pltpu.VMEM((2, _CH), jnp.int32),
        pltpu.VMEM((2, _CH, _D), jnp.float32),
        pltpu.VMEM((_CH + _L,), jnp.int32),
        pltpu.VMEM((_TAIL,), jnp.int32),
        pltpu.VMEM((_TAIL, _D), jnp.float32),
        pltpu.SemaphoreType.DMA,
        pltpu.SemaphoreType.DMA,
        pltpu.SemaphoreType.DMA,
        pltpu.SemaphoreType.DMA,
        pltpu.SemaphoreType.DMA,
        pltpu.SemaphoreType.DMA,
        pltpu.SemaphoreType.DMA,
    ],
)
def _sc_gather(x_hbm, idx_hbm, out_hbm, idx_v, rows_v, idx_s, idx_t, rows_t,
               si0, si1, sg0, sg1, sw0, sw1, st):
    wid = lax.axis_index("s") * 2 + lax.axis_index("c")
    nc = jnp.where(wid <= 12, 25, 24)

    sem_i = (si0, si1)
    sem_g = (sg0, sg1)
    sem_w = (sw0, sw1)

    def chunk_step(c, b):
        chunk = wid + c * _NW
        base = chunk * _CH
        my_idx = idx_v.at[b]
        my_rows = rows_v.at[b]

        @pl.when(c == 0)
        def _():
            pltpu.async_copy(idx_hbm.at[pl.ds(base, _CH)], my_idx, sem_i[b])
            pltpu.async_copy(
                idx_hbm.at[pl.ds(base + _NW * _CH, _CH)], idx_v.at[1 - b],
                sem_i[1 - b])

        pltpu.make_async_copy(idx_hbm.at[pl.ds(0, _CH)], my_idx,
                              sem_i[b]).wait()

        bad = _clamp_detect(my_idx, idx_s, _CH)

        @pl.when(c >= 2)
        def _():
            pltpu.make_async_copy(my_rows, out_hbm.at[pl.ds(0, _CH)],
                                  sem_w[b]).wait()

        gather = pltpu.async_copy(x_hbm.at[my_idx], my_rows, sem_g[b])
        gather.wait()

        @pl.when(bad)
        def _():
            _zero_shadow_rows(idx_s, my_rows, _CH)

        @pl.when(c + 2 < nc)
        def _():
            pltpu.async_copy(
                idx_hbm.at[pl.ds(base + 2 * _NW * _CH, _CH)], my_idx,
                sem_i[b])

        pltpu.async_copy(my_rows, out_hbm.at[pl.ds(base, _CH)], sem_w[b])

    def pair_body(p, carry):
        chunk_step(2 * p, 0)
        chunk_step(2 * p + 1, 1)
        return carry

    lax.fori_loop(0, _NPAIRS, pair_body, 0)

    @pl.when(nc == 25)
    def _():
        chunk_step(jnp.int32(24), 0)

    pltpu.make_async_copy(rows_v.at[0], out_hbm.at[pl.ds(0, _CH)], sw0).wait()
    pltpu.make_async_copy(rows_v.at[1], out_hbm.at[pl.ds(0, _CH)], sw1).wait()

    @pl.when(wid == _NW - 1)
    def _():
        tbase = _NFULL * _CH
        pltpu.sync_copy(idx_hbm.at[pl.ds(tbase, _TAIL)], idx_t)
        tbad = _clamp_detect(idx_t, idx_s, _TAIL)
        pltpu.async_copy(x_hbm.at[idx_t], rows_t, st).wait()

        @pl.when(tbad)
        def _():
            _zero_shadow_rows(idx_s, rows_t, _TAIL)

        pltpu.sync_copy(rows_t, out_hbm.at[pl.ds(tbase, _TAIL)])


def kernel(x, upsamples):
    return _sc_gather(x, upsamples[:, 0])
